# baseline (speedup 1.0000x reference)
import functools

import jax
import jax.numpy as jnp
from jax import lax
from jax.experimental import pallas as pl
from jax.experimental.pallas import tpu as pltpu
from jax.experimental.pallas import tpu_sc as plsc

_N = 50000
_E = 800000
_H = 32
_DN = 16
_DE = 4
_DEXT = 4
_DOUT = 3

_NC = 2
_NS = 16
_NW = _NC * _NS
_CH = 128
_NCHUNK = _E // _CH
_TMAX = -(-_NCHUNK // _NW)
_RPT = _N // _NS
_STG = 125

_BN = 2000
_BE = 8000


def _ln(h, g, b):
    mu = jnp.mean(h, axis=-1, keepdims=True)
    var = jnp.mean(jnp.square(h - mu), axis=-1, keepdims=True)
    return (h - mu) * lax.rsqrt(var + 1e-5) * g + b


def _mlp_ln(x, w1, b1, w2, b2, g, be):
    h = jnp.dot(x, w1, preferred_element_type=jnp.float32) + b1
    h = jnp.maximum(h, 0.0)
    h = jnp.dot(h, w2, preferred_element_type=jnp.float32) + b2
    return _ln(h, g, be)


def _dot(a, b):
    return jnp.dot(a, b, preferred_element_type=jnp.float32)


def _k1_body(x_ref, v_ref, sx_ref,
             nw1, nb1, nw2, nb2, ng, nbe,
             sw1, sb1, sw2, sb2, sg, sbe,
             wxi, wxj,
             encx_o, b_o, p_o, q_o):
    encx = _mlp_ln(x_ref[...], nw1[...], nb1[...], nw2[...], nb2[...],
                   ng[...], nbe[...])
    senc = _mlp_ln(sx_ref[...], sw1[...], sb1[...], sw2[...], sb2[...],
                   sg[...], sbe[...])
    bb = _dot(v_ref[...], senc)
    encx_o[...] = encx
    b_o[...] = bb
    p_o[...] = _dot(encx, wxi[0:32, :]) + _dot(bb, wxi[32:64, :])
    q_o[...] = _dot(encx, wxj[0:32, :]) + _dot(bb, wxj[32:64, :])


def _run_k1(x, V, spc_x, ne, se, w1):
    nblk = _N // _BN
    full = lambda shape: pl.BlockSpec(shape, lambda i: (0, 0))
    outs = jax.ShapeDtypeStruct((_N, _H), jnp.float32)
    return pl.pallas_call(
        _k1_body,
        grid=(nblk,),
        in_specs=[
            pl.BlockSpec((_BN, _DN), lambda i: (i, 0)),
            pl.BlockSpec((_BN, _H), lambda i: (i, 0)),
            full((32, _DN)),
            full((_DN, _H)), full((1, _H)), full((_H, _H)), full((1, _H)),
            full((1, _H)), full((1, _H)),
            full((_DN, _H)), full((1, _H)), full((_H, _H)), full((1, _H)),
            full((1, _H)), full((1, _H)),
            full((2 * _H, _H)), full((2 * _H, _H)),
        ],
        out_specs=[pl.BlockSpec((_BN, _H), lambda i: (i, 0))] * 4,
        out_shape=[outs, outs, outs, outs],
    )(x, V, spc_x,
      ne['w1'], ne['b1'].reshape(1, _H), ne['w2'], ne['b2'].reshape(1, _H),
      ne['g'].reshape(1, _H), ne['beta'].reshape(1, _H),
      se['w1'], se['b1'].reshape(1, _H), se['w2'], se['b2'].reshape(1, _H),
      se['g'].reshape(1, _H), se['beta'].reshape(1, _H),
      w1[0:64, :], w1[64:128, :])


_NSTEADY = _NCHUNK // _NW
_NTAIL = _NCHUNK - _NSTEADY * _NW


def _sc_gather_add(p, q, dst, src):
    mesh = plsc.VectorSubcoreMesh(core_axis_name="c", subcore_axis_name="s")

    @functools.partial(
        pl.kernel,
        out_type=jax.ShapeDtypeStruct((_E // 4, 128), jnp.float32),
        mesh=mesh,
        compiler_params=pltpu.CompilerParams(use_tc_tiling_on_sc=False),
        scratch_types=[
            pltpu.VMEM((_CH,), jnp.int32), pltpu.VMEM((_CH,), jnp.int32),
            pltpu.VMEM((_CH,), jnp.int32), pltpu.VMEM((_CH,), jnp.int32),
            pltpu.VMEM((_CH, _H), jnp.float32),
            pltpu.VMEM((_CH, _H), jnp.float32),
            pltpu.VMEM((_CH, _H), jnp.float32),
            pltpu.VMEM((_CH, _H), jnp.float32),
            pltpu.VMEM((_CH // 4, 128), jnp.float32),
            pltpu.VMEM((_CH // 4, 128), jnp.float32),
            pltpu.SemaphoreType.DMA, pltpu.SemaphoreType.DMA,
            pltpu.SemaphoreType.DMA, pltpu.SemaphoreType.DMA,
        ],
    )
    def k(p_h, q_h, dst_h, src_h, s_h,
          idxd0, idxd1, idxs0, idxs1, pb0, pb1, qb0, qb1, sb0, sb1,
          gsem0, gsem1, ssem0, ssem1):
        cid0 = lax.axis_index("s") * _NC + lax.axis_index("c")
        idxd = (idxd0, idxd1)
        idxs = (idxs0, idxs1)
        pb = (pb0, pb1)
        qb = (qb0, qb1)
        sb = (sb0, sb1)
        gsem = (gsem0, gsem1)
        ssem = (ssem0, ssem1)

        def fire(c, b):
            base = (cid0 + c * _NW) * _CH
            pltpu.sync_copy(dst_h.at[pl.ds(base, _CH)], idxd[b])
            pltpu.sync_copy(src_h.at[pl.ds(base, _CH)], idxs[b])
            pltpu.async_copy(p_h.at[idxd[b]], pb[b], gsem[b])
            pltpu.async_copy(q_h.at[idxs[b]], qb[b], gsem[b])

        def drain_gather(b):
            pltpu.make_async_copy(p_h.at[pl.ds(0, _CH)], pb[b], gsem[b]).wait()
            pltpu.make_async_copy(q_h.at[pl.ds(0, _CH)], qb[b], gsem[b]).wait()

        def add_rows(b):
            def row4(r4, c2):
                for kk in range(4):
                    for hh in (0, 16):
                        sb[b][r4, pl.ds(kk * 32 + hh, 16)] = (
                            pb[b][r4 * 4 + kk, pl.ds(hh, 16)]
                            + qb[b][r4 * 4 + kk, pl.ds(hh, 16)])
                return c2

            lax.fori_loop(0, _CH // 4, row4, 0, unroll=4)

        def store_async(c, b):
            base4 = (cid0 + c * _NW) * (_CH // 4)
            pltpu.async_copy(sb[b], s_h.at[pl.ds(base4, _CH // 4)], ssem[b])

        def drain_store(b):
            pltpu.make_async_copy(sb[b], s_h.at[pl.ds(0, _CH // 4)],
                                  ssem[b]).wait()

        fire(0, 0)

        def body(i, carry):
            @pl.when(i > 0)
            def _():
                drain_store(1)
            fire(2 * i + 1, 1)
            drain_gather(0)
            add_rows(0)
            store_async(2 * i, 0)
            drain_store(0)
            fire(2 * i + 2, 0)
            drain_gather(1)
            add_rows(1)
            store_async(2 * i + 1, 1)
            return carry

        lax.fori_loop(0, (_NSTEADY - 1) // 2, body, 0)
        drain_gather(0)
        add_rows(0)
        base4_l = (cid0 + (_NSTEADY - 1) * _NW) * (_CH // 4)
        pltpu.sync_copy(sb[0], s_h.at[pl.ds(base4_l, _CH // 4)])
        drain_store(1)

        @pl.when(cid0 < _NTAIL)
        def _():
            base = (cid0 + _NSTEADY * _NW) * _CH
            pltpu.sync_copy(dst_h.at[pl.ds(base, _CH)], idxd[0])
            pltpu.sync_copy(src_h.at[pl.ds(base, _CH)], idxs[0])
            d1 = pltpu.async_copy(p_h.at[idxd[0]], pb[0], gsem[0])
            d2 = pltpu.async_copy(q_h.at[idxs[0]], qb[0], gsem[1])
            d1.wait()
            d2.wait()
            add_rows(0)
            pltpu.sync_copy(sb[0], s_h.at[pl.ds(base // 4, _CH // 4)])

    return k(p, q, dst, src)


_BR = _BE // 4


def _bd4(w):
    z = jnp.zeros_like(w)
    return jnp.block([[w, z, z, z], [z, w, z, z], [z, z, w, z], [z, z, z, w]])


def _t4(v):
    return jnp.tile(v.reshape(1, _H), (1, 4))


def _ln_g(h, avg, g, be):
    mu = _dot(h, avg)
    d = h - mu
    var = _dot(d * d, avg)
    return d * lax.rsqrt(var + 1e-5) * g + be


def _k3_body(s_ref, ea_ref,
             m1, eb1, ew2, eb2, eg, ebe,
             w1e, pb1, pw2, pb2, pg, pbe, avg,
             ue_o):
    e1 = jnp.maximum(_dot(ea_ref[...], m1[...]) + eb1[...], 0.0)
    av = avg[...]
    eenc = _ln_g(_dot(e1, ew2[...]) + eb2[...], av, eg[...], ebe[...])
    h = s_ref[...] + _dot(eenc, w1e[...]) + pb1[...]
    h = jnp.maximum(h, 0.0)
    h = _dot(h, pw2[...]) + pb2[...]
    ue_o[...] = _ln_g(h, av, pg[...], pbe[...]) + eenc


def _run_k3(sp, edge_attr, ee, pm):
    nblk = _E // _BE
    ea16 = edge_attr.reshape(_E // 4, 16)
    full = lambda shape: pl.BlockSpec(shape, lambda i: (0, 0))
    avg = _bd4(jnp.full((_H, _H), 1.0 / _H, jnp.float32))
    ue = pl.pallas_call(
        _k3_body,
        grid=(nblk,),
        in_specs=[
            pl.BlockSpec((_BR, 128), lambda i: (i, 0)),
            pl.BlockSpec((_BR, 16), lambda i: (i, 0)),
            full((16, 128)), full((1, 128)), full((128, 128)), full((1, 128)),
            full((1, 128)), full((1, 128)),
            full((128, 128)), full((1, 128)), full((128, 128)),
            full((1, 128)), full((1, 128)), full((1, 128)), full((128, 128)),
        ],
        out_specs=pl.BlockSpec((_BR, 128), lambda i: (i, 0)),
        out_shape=jax.ShapeDtypeStruct((_E // 4, 128), jnp.float32),
    )(sp, ea16,
      _bd4(ee['w1']), _t4(ee['b1']), _bd4(ee['w2']), _t4(ee['b2']),
      _t4(ee['g']), _t4(ee['beta']),
      _bd4(pm['w1'][128:160, :]), _t4(pm['b1']), _bd4(pm['w2']),
      _t4(pm['b2']), _t4(pm['g']), _t4(pm['beta']), avg)
    return ue


def _sc_scatter_add(ue, src, zrows):
    mesh = plsc.VectorSubcoreMesh(core_axis_name="c", subcore_axis_name="s")

    @functools.partial(
        pl.kernel,
        out_type=jax.ShapeDtypeStruct((_NC, _N, _H), jnp.float32),
        mesh=mesh,
        compiler_params=pltpu.CompilerParams(use_tc_tiling_on_sc=False),
        scratch_types=[
            pltpu.VMEM((_CH,), jnp.int32), pltpu.VMEM((_CH,), jnp.int32),
            pltpu.VMEM((_CH // 4, 128), jnp.float32),
            pltpu.VMEM((_CH // 4, 128), jnp.float32),
            pltpu.VMEM((_CH, _H), jnp.float32),
            pltpu.VMEM((_STG, _H), jnp.float32),
            pltpu.VMEM_SHARED((_N, _H), jnp.float32),
            pltpu.SemaphoreType.DMA, pltpu.SemaphoreType.DMA,
        ],
    )
    def k(ue_h, src_h, z_h, out_h, idxv0, idxv1, ueb0, ueb1, ub32, stage,
          accum, lsem0, lsem1):
        c = lax.axis_index("c")
        s = lax.axis_index("s")
        cid0 = s * _NC + c
        idxv = (idxv0, idxv1)
        ueb = (ueb0, ueb1)
        lsem = (lsem0, lsem1)

        pltpu.sync_copy(z_h, stage)
        for j in range(_RPT // _STG):
            pltpu.sync_copy(stage, accum.at[pl.ds(s * _RPT + j * _STG, _STG)])
        plsc.subcore_barrier()

        def fire(t, b):
            base = (cid0 + t * _NW) * _CH
            pltpu.async_copy(src_h.at[pl.ds(base, _CH)], idxv[b], lsem[b])
            pltpu.async_copy(ue_h.at[pl.ds(base // 4, _CH // 4)], ueb[b],
                             lsem[b])

        def drain(b):
            pltpu.make_async_copy(src_h.at[pl.ds(0, _CH)], idxv[b],
                                  lsem[b]).wait()
            pltpu.make_async_copy(ue_h.at[pl.ds(0, _CH // 4)], ueb[b],
                                  lsem[b]).wait()

        def scat(b):
            def row4(r4, c2):
                for kk in range(4):
                    for hh in (0, 16):
                        ub32[r4 * 4 + kk, pl.ds(hh, 16)] = (
                            ueb[b][r4, pl.ds(kk * 32 + hh, 16)])
                return c2

            lax.fori_loop(0, _CH // 4, row4, 0, unroll=4)
            pltpu.sync_copy(ub32, accum.at[idxv[b]], add=True)

        fire(0, 0)

        def body(i, carry):
            fire(2 * i + 1, 1)
            drain(0)
            scat(0)
            fire(2 * i + 2, 0)
            drain(1)
            scat(1)
            return carry

        lax.fori_loop(0, (_NSTEADY - 1) // 2, body, 0)
        drain(0)
        scat(0)

        @pl.when(cid0 < _NTAIL)
        def _():
            base = (cid0 + _NSTEADY * _NW) * _CH
            pltpu.sync_copy(src_h.at[pl.ds(base, _CH)], idxv[0])
            pltpu.sync_copy(ue_h.at[pl.ds(base // 4, _CH // 4)], ueb[0])
            scat(0)

        plsc.subcore_barrier()
        for j in range(_RPT // _STG):
            row0 = s * _RPT + j * _STG
            pltpu.sync_copy(accum.at[pl.ds(row0, _STG)], stage)
            pltpu.sync_copy(stage, out_h.at[c, pl.ds(row0, _STG)])

    return k(ue, src, zrows)


def _k5_body(encx_ref, b_ref, p0_ref, p1_ref, dx_ref,
             w1a, w1b, w1c, w1d, nb1, nw2, nb2, ng, nbe,
             dw1a, dw1b, db1, dw2, db2,
             out_o):
    encx = encx_ref[...]
    bb = b_ref[...]
    agg = p0_ref[...] + p1_ref[...]
    h = (_dot(encx, w1a[...]) + _dot(bb, w1b[...]) + _dot(agg, w1c[...])
         + _dot(dx_ref[...], w1d[...]) + nb1[...])
    h = jnp.maximum(h, 0.0)
    h = _dot(h, nw2[...]) + nb2[...]
    xp = encx + _ln(h, ng[...], nbe[...])
    d = jnp.maximum(_dot(xp, dw1a[...]) + _dot(bb, dw1b[...]) + db1[...], 0.0)
    out_o[...] = _dot(d, dw2[...]) + db2[...]


def _run_k5(encx, bb, p0, p1, dext, nm, dec):
    nblk = _N // _BN
    full = lambda shape: pl.BlockSpec(shape, lambda i: (0, 0))
    w1 = nm['w1']
    return pl.pallas_call(
        _k5_body,
        grid=(nblk,),
        in_specs=[
            pl.BlockSpec((_BN, _H), lambda i: (i, 0)),
            pl.BlockSpec((_BN, _H), lambda i: (i, 0)),
            pl.BlockSpec((_BN, _H), lambda i: (i, 0)),
            pl.BlockSpec((_BN, _H), lambda i: (i, 0)),
            pl.BlockSpec((_BN, _DEXT), lambda i: (i, 0)),
            full((_H, _H)), full((_H, _H)), full((_H, _H)), full((_DEXT, _H)),
            full((1, _H)), full((_H, _H)), full((1, _H)), full((1, _H)),
            full((1, _H)),
            full((_H, _H)), full((_H, _H)), full((1, _H)), full((_H, _DOUT)),
            full((1, _DOUT)),
        ],
        out_specs=pl.BlockSpec((_BN, _DOUT), lambda i: (i, 0)),
        out_shape=jax.ShapeDtypeStruct((_N, _DOUT), jnp.float32),
    )(encx, bb, p0, p1, dext,
      w1[0:32, :], w1[32:64, :], w1[64:96, :], w1[96:100, :],
      nm['b1'].reshape(1, _H), nm['w2'], nm['b2'].reshape(1, _H),
      nm['g'].reshape(1, _H), nm['beta'].reshape(1, _H),
      dec['w1'][0:32, :], dec['w1'][32:64, :], dec['b1'].reshape(1, _H),
      dec['w2'], dec['b2'].reshape(1, _DOUT))


def kernel(x, edge_index, edge_attr, del_ext_force, spc_x, spc_edge_index,
           spc_edge_attr, V, params):
    src = edge_index[0]
    dst = edge_index[1]
    pm = params['processor'][0]['edge_mlp']
    encx, bb, p, q = _run_k1(x, V, spc_x, params['node_encoder'],
                             params['spc_node_encoder'], pm['w1'])
    s = _sc_gather_add(p, q, dst, src)
    ue = _run_k3(s, edge_attr, params['edge_encoder'], pm)
    parts = _sc_scatter_add(ue, src, jnp.zeros((_STG, _H), jnp.float32))
    out = _run_k5(encx, bb, parts[0], parts[1], del_ext_force,
                  params['processor'][0]['node_mlp'], params['decoder'])
    return out


# speedup vs baseline: 1.3747x; 1.3747x over previous
import functools

import jax
import jax.numpy as jnp
from jax import lax
from jax.experimental import pallas as pl
from jax.experimental.pallas import tpu as pltpu
from jax.experimental.pallas import tpu_sc as plsc

_N = 50000
_E = 800000
_H = 32
_DN = 16
_DE = 4
_DEXT = 4
_DOUT = 3

_NC = 2
_NS = 16
_NW = _NC * _NS
_CH = 128
_NCHUNK = _E // _CH
_TMAX = -(-_NCHUNK // _NW)
_RPT = _N // _NS
_STG = 125

_BN = 2000
_BE = 8000


def _ln(h, g, b):
    mu = jnp.mean(h, axis=-1, keepdims=True)
    var = jnp.mean(jnp.square(h - mu), axis=-1, keepdims=True)
    return (h - mu) * lax.rsqrt(var + 1e-5) * g + b


def _mlp_ln(x, w1, b1, w2, b2, g, be):
    h = jnp.dot(x, w1, preferred_element_type=jnp.float32) + b1
    h = jnp.maximum(h, 0.0)
    h = jnp.dot(h, w2, preferred_element_type=jnp.float32) + b2
    return _ln(h, g, be)


def _dot(a, b):
    return jnp.dot(a, b, preferred_element_type=jnp.float32)


def _k1_body(x_ref, v_ref, sx_ref,
             nw1, nb1, nw2, nb2, ng, nbe,
             sw1, sb1, sw2, sb2, sg, sbe,
             wxi, wxj,
             encx_o, b_o, p_o, q_o):
    encx = _mlp_ln(x_ref[...], nw1[...], nb1[...], nw2[...], nb2[...],
                   ng[...], nbe[...])
    senc = _mlp_ln(sx_ref[...], sw1[...], sb1[...], sw2[...], sb2[...],
                   sg[...], sbe[...])
    bb = _dot(v_ref[...], senc)
    encx_o[...] = encx
    b_o[...] = bb
    p_o[...] = _dot(encx, wxi[0:32, :]) + _dot(bb, wxi[32:64, :])
    q_o[...] = _dot(encx, wxj[0:32, :]) + _dot(bb, wxj[32:64, :])


def _run_k1(x, V, spc_x, ne, se, w1):
    nblk = _N // _BN
    full = lambda shape: pl.BlockSpec(shape, lambda i: (0, 0))
    outs = jax.ShapeDtypeStruct((_N, _H), jnp.float32)
    return pl.pallas_call(
        _k1_body,
        grid=(nblk,),
        in_specs=[
            pl.BlockSpec((_BN, _DN), lambda i: (i, 0)),
            pl.BlockSpec((_BN, _H), lambda i: (i, 0)),
            full((32, _DN)),
            full((_DN, _H)), full((1, _H)), full((_H, _H)), full((1, _H)),
            full((1, _H)), full((1, _H)),
            full((_DN, _H)), full((1, _H)), full((_H, _H)), full((1, _H)),
            full((1, _H)), full((1, _H)),
            full((2 * _H, _H)), full((2 * _H, _H)),
        ],
        out_specs=[pl.BlockSpec((_BN, _H), lambda i: (i, 0))] * 4,
        out_shape=[outs, outs, outs, outs],
    )(x, V, spc_x,
      ne['w1'], ne['b1'].reshape(1, _H), ne['w2'], ne['b2'].reshape(1, _H),
      ne['g'].reshape(1, _H), ne['beta'].reshape(1, _H),
      se['w1'], se['b1'].reshape(1, _H), se['w2'], se['b2'].reshape(1, _H),
      se['g'].reshape(1, _H), se['beta'].reshape(1, _H),
      w1[0:64, :], w1[64:128, :])


_NSTEADY = _NCHUNK // _NW
_NTAIL = _NCHUNK - _NSTEADY * _NW


def _sc_gather_add(p, q, dst, src):
    mesh = plsc.VectorSubcoreMesh(core_axis_name="c", subcore_axis_name="s")

    @functools.partial(
        pl.kernel,
        out_type=jax.ShapeDtypeStruct((_E, _H), jnp.float32),
        mesh=mesh,
        compiler_params=pltpu.CompilerParams(use_tc_tiling_on_sc=False),
        scratch_types=[
            pltpu.VMEM((_CH,), jnp.int32), pltpu.VMEM((_CH,), jnp.int32),
            pltpu.VMEM((_CH,), jnp.int32), pltpu.VMEM((_CH,), jnp.int32),
            pltpu.VMEM((_CH, _H), jnp.float32),
            pltpu.VMEM((_CH, _H), jnp.float32),
            pltpu.VMEM((_CH, _H), jnp.float32),
            pltpu.VMEM((_CH, _H), jnp.float32),
            pltpu.SemaphoreType.DMA, pltpu.SemaphoreType.DMA,
            pltpu.SemaphoreType.DMA, pltpu.SemaphoreType.DMA,
        ],
    )
    def k(p_h, q_h, dst_h, src_h, s_h,
          idxd0, idxd1, idxs0, idxs1, pb0, pb1, qb0, qb1,
          gsem0, gsem1, ssem0, ssem1):
        cid0 = lax.axis_index("s") * _NC + lax.axis_index("c")
        idxd = (idxd0, idxd1)
        idxs = (idxs0, idxs1)
        pb = (pb0, pb1)
        qb = (qb0, qb1)
        gsem = (gsem0, gsem1)
        ssem = (ssem0, ssem1)

        def fire(c, b):
            base = (cid0 + c * _NW) * _CH
            pltpu.sync_copy(dst_h.at[pl.ds(base, _CH)], idxd[b])
            pltpu.sync_copy(src_h.at[pl.ds(base, _CH)], idxs[b])
            pltpu.async_copy(p_h.at[idxd[b]], pb[b], gsem[b])
            pltpu.async_copy(q_h.at[idxs[b]], qb[b], gsem[b])

        def drain_gather(b):
            pltpu.make_async_copy(p_h.at[pl.ds(0, _CH)], pb[b], gsem[b]).wait()
            pltpu.make_async_copy(q_h.at[pl.ds(0, _CH)], qb[b], gsem[b]).wait()

        def add_rows(b):
            def row(r, c2):
                pb[b][r, pl.ds(0, 16)] = (pb[b][r, pl.ds(0, 16)]
                                          + qb[b][r, pl.ds(0, 16)])
                pb[b][r, pl.ds(16, 16)] = (pb[b][r, pl.ds(16, 16)]
                                           + qb[b][r, pl.ds(16, 16)])
                return c2

            lax.fori_loop(0, _CH, row, 0, unroll=8)

        def store_async(c, b):
            base = (cid0 + c * _NW) * _CH
            pltpu.async_copy(pb[b], s_h.at[pl.ds(base, _CH)], ssem[b])

        def drain_store(b):
            pltpu.make_async_copy(pb[b], s_h.at[pl.ds(0, _CH)],
                                  ssem[b]).wait()

        fire(0, 0)

        def body(i, carry):
            @pl.when(i > 0)
            def _():
                drain_store(1)
            fire(2 * i + 1, 1)
            drain_gather(0)
            add_rows(0)
            store_async(2 * i, 0)
            drain_store(0)
            fire(2 * i + 2, 0)
            drain_gather(1)
            add_rows(1)
            store_async(2 * i + 1, 1)
            return carry

        lax.fori_loop(0, (_NSTEADY - 1) // 2, body, 0)
        drain_gather(0)
        add_rows(0)
        base_l = (cid0 + (_NSTEADY - 1) * _NW) * _CH
        pltpu.sync_copy(pb[0], s_h.at[pl.ds(base_l, _CH)])
        drain_store(1)

        @pl.when(cid0 < _NTAIL)
        def _():
            base = (cid0 + _NSTEADY * _NW) * _CH
            pltpu.sync_copy(dst_h.at[pl.ds(base, _CH)], idxd[0])
            pltpu.sync_copy(src_h.at[pl.ds(base, _CH)], idxs[0])
            d1 = pltpu.async_copy(p_h.at[idxd[0]], pb[0], gsem[0])
            d2 = pltpu.async_copy(q_h.at[idxs[0]], qb[0], gsem[1])
            d1.wait()
            d2.wait()
            add_rows(0)
            pltpu.sync_copy(pb[0], s_h.at[pl.ds(base, _CH)])

    return k(p, q, dst, src)


_BR = _BE // 4


def _bd4(w):
    z = jnp.zeros_like(w)
    return jnp.block([[w, z, z, z], [z, w, z, z], [z, z, w, z], [z, z, z, w]])


def _t4(v):
    return jnp.tile(v.reshape(1, _H), (1, 4))


def _ln_g(h, avg, g, be):
    mu = _dot(h, avg)
    d = h - mu
    var = _dot(d * d, avg)
    return d * lax.rsqrt(var + 1e-5) * g + be


def _k0e_body(ea_ref, ew1, eb1, e1_o):
    e1_o[...] = jnp.maximum(_dot(ea_ref[...], ew1[...]) + eb1[...], 0.0)


def _run_k0e(edge_attr, ee):
    nblk = _E // _BE
    full = lambda shape: pl.BlockSpec(shape, lambda i: (0, 0))
    return pl.pallas_call(
        _k0e_body,
        grid=(nblk,),
        in_specs=[
            pl.BlockSpec((_BE, _DE), lambda i: (i, 0)),
            full((_DE, _H)), full((1, _H)),
        ],
        out_specs=pl.BlockSpec((_BE, _H), lambda i: (i, 0)),
        out_shape=jax.ShapeDtypeStruct((_E, _H), jnp.float32),
    )(edge_attr, ee['w1'], ee['b1'].reshape(1, _H))


def _k3_body(s_ref, e1_ref,
             ew2, eb2, eg, ebe,
             w1e, pb1, pw2, pb2, pg, pbe, avg,
             ue_o):
    av = avg[...]
    eenc = _ln_g(_dot(e1_ref[...], ew2[...]) + eb2[...], av, eg[...],
                 ebe[...])
    h = s_ref[...] + _dot(eenc, w1e[...]) + pb1[...]
    h = jnp.maximum(h, 0.0)
    h = _dot(h, pw2[...]) + pb2[...]
    ue_o[...] = _ln_g(h, av, pg[...], pbe[...]) + eenc


def _run_k3(sp, e1p, ee, pm):
    nblk = _E // _BE
    full = lambda shape: pl.BlockSpec(shape, lambda i: (0, 0))
    avg = _bd4(jnp.full((_H, _H), 1.0 / _H, jnp.float32))
    return pl.pallas_call(
        _k3_body,
        grid=(nblk,),
        in_specs=[
            pl.BlockSpec((_BR, 128), lambda i: (i, 0)),
            pl.BlockSpec((_BR, 128), lambda i: (i, 0)),
            full((128, 128)), full((1, 128)),
            full((1, 128)), full((1, 128)),
            full((128, 128)), full((1, 128)), full((128, 128)),
            full((1, 128)), full((1, 128)), full((1, 128)), full((128, 128)),
        ],
        out_specs=pl.BlockSpec((_BR, 128), lambda i: (i, 0)),
        out_shape=jax.ShapeDtypeStruct((_E // 4, 128), jnp.float32),
    )(sp, e1p,
      _bd4(ee['w2']), _t4(ee['b2']),
      _t4(ee['g']), _t4(ee['beta']),
      _bd4(pm['w1'][128:160, :]), _t4(pm['b1']), _bd4(pm['w2']),
      _t4(pm['b2']), _t4(pm['g']), _t4(pm['beta']), avg)


def _sc_scatter_add(ue, src, zrows):
    mesh = plsc.VectorSubcoreMesh(core_axis_name="c", subcore_axis_name="s")

    @functools.partial(
        pl.kernel,
        out_type=jax.ShapeDtypeStruct((_NC, _N, _H), jnp.float32),
        mesh=mesh,
        compiler_params=pltpu.CompilerParams(use_tc_tiling_on_sc=False),
        scratch_types=[
            pltpu.VMEM((_CH,), jnp.int32), pltpu.VMEM((_CH,), jnp.int32),
            pltpu.VMEM((_CH, _H), jnp.float32),
            pltpu.VMEM((_CH, _H), jnp.float32),
            pltpu.VMEM((_STG, _H), jnp.float32),
            pltpu.VMEM_SHARED((_N, _H), jnp.float32),
            pltpu.SemaphoreType.DMA, pltpu.SemaphoreType.DMA,
        ],
    )
    def k(ue_h, src_h, z_h, out_h, idxv0, idxv1, ueb0, ueb1, stage,
          accum, lsem0, lsem1):
        c = lax.axis_index("c")
        s = lax.axis_index("s")
        cid0 = s * _NC + c
        idxv = (idxv0, idxv1)
        ueb = (ueb0, ueb1)
        lsem = (lsem0, lsem1)

        pltpu.sync_copy(z_h, stage)
        for j in range(_RPT // _STG):
            pltpu.sync_copy(stage, accum.at[pl.ds(s * _RPT + j * _STG, _STG)])
        plsc.subcore_barrier()

        def fire(t, b):
            base = (cid0 + t * _NW) * _CH
            pltpu.async_copy(src_h.at[pl.ds(base, _CH)], idxv[b], lsem[b])
            pltpu.async_copy(ue_h.at[pl.ds(base, _CH)], ueb[b], lsem[b])

        def drain(b):
            pltpu.make_async_copy(src_h.at[pl.ds(0, _CH)], idxv[b],
                                  lsem[b]).wait()
            pltpu.make_async_copy(ue_h.at[pl.ds(0, _CH)], ueb[b],
                                  lsem[b]).wait()

        def scat(b):
            pltpu.sync_copy(ueb[b], accum.at[idxv[b]], add=True)

        fire(0, 0)

        def body(i, carry):
            fire(2 * i + 1, 1)
            drain(0)
            scat(0)
            fire(2 * i + 2, 0)
            drain(1)
            scat(1)
            return carry

        lax.fori_loop(0, (_NSTEADY - 1) // 2, body, 0)
        drain(0)
        scat(0)

        @pl.when(cid0 < _NTAIL)
        def _():
            base = (cid0 + _NSTEADY * _NW) * _CH
            pltpu.sync_copy(src_h.at[pl.ds(base, _CH)], idxv[0])
            pltpu.sync_copy(ue_h.at[pl.ds(base, _CH)], ueb[0])
            scat(0)

        plsc.subcore_barrier()
        for j in range(_RPT // _STG):
            row0 = s * _RPT + j * _STG
            pltpu.sync_copy(accum.at[pl.ds(row0, _STG)], stage)
            pltpu.sync_copy(stage, out_h.at[c, pl.ds(row0, _STG)])

    return k(ue, src, zrows)


def _k5_body(encx_ref, b_ref, p0_ref, p1_ref, dx_ref,
             w1a, w1b, w1c, w1d, nb1, nw2, nb2, ng, nbe,
             dw1a, dw1b, db1, dw2, db2,
             out_o):
    encx = encx_ref[...]
    bb = b_ref[...]
    agg = p0_ref[...] + p1_ref[...]
    h = (_dot(encx, w1a[...]) + _dot(bb, w1b[...]) + _dot(agg, w1c[...])
         + _dot(dx_ref[...], w1d[...]) + nb1[...])
    h = jnp.maximum(h, 0.0)
    h = _dot(h, nw2[...]) + nb2[...]
    xp = encx + _ln(h, ng[...], nbe[...])
    d = jnp.maximum(_dot(xp, dw1a[...]) + _dot(bb, dw1b[...]) + db1[...], 0.0)
    out_o[...] = _dot(d, dw2[...]) + db2[...]


def _run_k5(encx, bb, p0, p1, dext, nm, dec):
    nblk = _N // _BN
    full = lambda shape: pl.BlockSpec(shape, lambda i: (0, 0))
    w1 = nm['w1']
    return pl.pallas_call(
        _k5_body,
        grid=(nblk,),
        in_specs=[
            pl.BlockSpec((_BN, _H), lambda i: (i, 0)),
            pl.BlockSpec((_BN, _H), lambda i: (i, 0)),
            pl.BlockSpec((_BN, _H), lambda i: (i, 0)),
            pl.BlockSpec((_BN, _H), lambda i: (i, 0)),
            pl.BlockSpec((_BN, _DEXT), lambda i: (i, 0)),
            full((_H, _H)), full((_H, _H)), full((_H, _H)), full((_DEXT, _H)),
            full((1, _H)), full((_H, _H)), full((1, _H)), full((1, _H)),
            full((1, _H)),
            full((_H, _H)), full((_H, _H)), full((1, _H)), full((_H, _DOUT)),
            full((1, _DOUT)),
        ],
        out_specs=pl.BlockSpec((_BN, _DOUT), lambda i: (i, 0)),
        out_shape=jax.ShapeDtypeStruct((_N, _DOUT), jnp.float32),
    )(encx, bb, p0, p1, dext,
      w1[0:32, :], w1[32:64, :], w1[64:96, :], w1[96:100, :],
      nm['b1'].reshape(1, _H), nm['w2'], nm['b2'].reshape(1, _H),
      nm['g'].reshape(1, _H), nm['beta'].reshape(1, _H),
      dec['w1'][0:32, :], dec['w1'][32:64, :], dec['b1'].reshape(1, _H),
      dec['w2'], dec['b2'].reshape(1, _DOUT))


def kernel(x, edge_index, edge_attr, del_ext_force, spc_x, spc_edge_index,
           spc_edge_attr, V, params):
    src = edge_index[0]
    dst = edge_index[1]
    pm = params['processor'][0]['edge_mlp']
    encx, bb, p, q = _run_k1(x, V, spc_x, params['node_encoder'],
                             params['spc_node_encoder'], pm['w1'])
    s = _sc_gather_add(p, q, dst, src)
    e1 = _run_k0e(edge_attr, params['edge_encoder'])
    ue = _run_k3(s.reshape(_E // 4, 128), e1.reshape(_E // 4, 128),
                 params['edge_encoder'], pm)
    parts = _sc_scatter_add(ue.reshape(_E, _H), src,
                            jnp.zeros((_STG, _H), jnp.float32))
    out = _run_k5(encx, bb, parts[0], parts[1], del_ext_force,
                  params['processor'][0]['node_mlp'], params['decoder'])
    return out
